# baseline (speedup 1.0000x reference)
import functools

import jax
import jax.numpy as jnp
from jax import lax
from jax.experimental import pallas as pl
from jax.experimental.pallas import tpu as pltpu
from jax.experimental.pallas import tpu_sc as plsc

N = 10000
NPAD = 10240
IN_DIM = 128
HID = 64
HHALF = HID // 2

NC = 2
NS = 16
NW = NC * NS
CHUNK = 128
ROWS_PER_TILE = NPAD // NS

E = 320000
MC = 160
EPAD = NS * MC * CHUNK
ED = E + N
DC = -(-ED // (NW * CHUNK))
EDPAD = NW * DC * CHUNK

_MESH = plsc.VectorSubcoreMesh(core_axis_name="c", subcore_axis_name="s")
_SC_PARAMS = pltpu.CompilerParams(use_tc_tiling_on_sc=False)

BM = 1024


def _deg_body(dcol_hbm, ones_hbm, zeros_hbm, out_hbm, idx_v, src_v, ztile_v,
              acc_sh, sem):
    c = lax.axis_index("c")
    s = lax.axis_index("s")
    w = s * NC + c
    pltpu.sync_copy(ones_hbm, src_v)
    pltpu.sync_copy(zeros_hbm, ztile_v)
    pltpu.sync_copy(dcol_hbm.at[w], idx_v)
    pltpu.sync_copy(ztile_v, acc_sh.at[pl.ds(s * ROWS_PER_TILE, ROWS_PER_TILE)])
    plsc.subcore_barrier()

    def body(j, carry):
        pltpu.sync_copy(src_v, acc_sh.at[idx_v.at[j]], add=True)
        return carry

    lax.fori_loop(0, DC, body, 0)
    plsc.subcore_barrier()
    pltpu.sync_copy(acc_sh.at[pl.ds(s * ROWS_PER_TILE, ROWS_PER_TILE)],
                    out_hbm.at[c, pl.ds(s * ROWS_PER_TILE, ROWS_PER_TILE)])


_deg_kernel = functools.partial(
    pl.kernel,
    out_type=jax.ShapeDtypeStruct((NC, NPAD, 16), jnp.float32),
    mesh=_MESH,
    compiler_params=_SC_PARAMS,
    scratch_types=[
        pltpu.VMEM((DC, CHUNK), jnp.int32),
        pltpu.VMEM((CHUNK, 16), jnp.float32),
        pltpu.VMEM((ROWS_PER_TILE, 16), jnp.float32),
        pltpu.VMEM_SHARED((NPAD, 16), jnp.float32),
        pltpu.SemaphoreType.DMA,
    ],
)(_deg_body)


_NBUF = 4


def _edge_body(g_hbm, ridx_hbm, cidx_hbm, zeros_hbm, out_hbm, ridx_v, cidx_v,
               bufs, ztile_v, g_sh, acc_sh, gsems, ssems):
    c = lax.axis_index("c")
    s = lax.axis_index("s")
    pltpu.sync_copy(ridx_hbm.at[s], ridx_v)
    pltpu.sync_copy(cidx_hbm.at[s], cidx_v)
    pltpu.sync_copy(zeros_hbm, ztile_v)
    pltpu.sync_copy(ztile_v, acc_sh.at[pl.ds(s * ROWS_PER_TILE, ROWS_PER_TILE)])
    pltpu.sync_copy(g_hbm.at[c, pl.ds(s * ROWS_PER_TILE, ROWS_PER_TILE)],
                    ztile_v)
    pltpu.sync_copy(ztile_v, g_sh.at[pl.ds(s * ROWS_PER_TILE, ROWS_PER_TILE)])
    plsc.subcore_barrier()

    nquad = MC // _NBUF
    for k in range(_NBUF):
        pltpu.async_copy(g_sh.at[ridx_v.at[k]], bufs[k], gsems[k])

    def body(i, carry):
        for k in range(_NBUF):
            j = _NBUF * i + k
            pltpu.make_async_copy(g_sh.at[ridx_v.at[j]], bufs[k],
                                  gsems[k]).wait()
            pltpu.async_copy(bufs[k], acc_sh.at[cidx_v.at[j]], ssems[k],
                             add=True)

        @pl.when(i < nquad - 1)
        def _():
            for k in range(_NBUF):
                j = _NBUF * i + k
                pltpu.make_async_copy(bufs[k], acc_sh.at[cidx_v.at[j]],
                                      ssems[k]).wait()
                pltpu.async_copy(g_sh.at[ridx_v.at[j + _NBUF]], bufs[k],
                                 gsems[k])

        return carry

    lax.fori_loop(0, nquad, body, 0)
    for k in range(_NBUF):
        pltpu.make_async_copy(bufs[k], acc_sh.at[cidx_v.at[MC - _NBUF + k]],
                              ssems[k]).wait()
    plsc.subcore_barrier()
    pltpu.sync_copy(acc_sh.at[pl.ds(s * ROWS_PER_TILE, ROWS_PER_TILE)],
                    out_hbm.at[c, pl.ds(s * ROWS_PER_TILE, ROWS_PER_TILE)])


_edge_kernel = functools.partial(
    pl.kernel,
    out_type=jax.ShapeDtypeStruct((NC, NPAD, HHALF), jnp.float32),
    mesh=_MESH,
    compiler_params=_SC_PARAMS,
    scratch_types=[
        pltpu.VMEM((MC, CHUNK), jnp.int32),
        pltpu.VMEM((MC, CHUNK), jnp.int32),
        [pltpu.VMEM((CHUNK, HHALF), jnp.float32) for _ in range(_NBUF)],
        pltpu.VMEM((ROWS_PER_TILE, HHALF), jnp.float32),
        pltpu.VMEM_SHARED((NPAD, HHALF), jnp.float32),
        pltpu.VMEM_SHARED((NPAD, HHALF), jnp.float32),
        [pltpu.SemaphoreType.DMA for _ in range(_NBUF)],
        [pltpu.SemaphoreType.DMA for _ in range(_NBUF)],
    ],
)(_edge_body)


def _dinv_block(degp_ref):
    deg = degp_ref[0, :, 0:1] + degp_ref[1, :, 0:1]
    return jnp.where(deg > 0.0, lax.rsqrt(deg), 0.0)


def _split_cols(g_ref, v):
    g_ref[0] = v[:, :HHALF]
    g_ref[1] = v[:, HHALF:]


def _join_cols(ref):
    return jnp.concatenate([ref[0], ref[1]], axis=1)


def _tc1_body(x_ref, w1_ref, degp_ref, g_ref):
    dinv = _dinv_block(degp_ref)
    h = jnp.dot(x_ref[...], w1_ref[...], preferred_element_type=jnp.float32)
    _split_cols(g_ref, h * dinv)


def _tc2_body(s_ref, g1_ref, degp_ref, w2_ref, b1_ref, g2_ref):
    dinv = _dinv_block(degp_ref)
    a1 = dinv * (_join_cols(s_ref) + _join_cols(g1_ref)) + b1_ref[...]
    r = jnp.maximum(a1, 0.0)
    h2 = jnp.dot(r, w2_ref[...], preferred_element_type=jnp.float32)
    _split_cols(g2_ref, h2 * dinv)


def _tc3_body(s_ref, g2_ref, degp_ref, wp_ref, b2_ref, bp_ref, z_ref):
    dinv = _dinv_block(degp_ref)
    a2 = dinv * (_join_cols(s_ref) + _join_cols(g2_ref)) + b2_ref[...]
    r = jnp.maximum(a2, 0.0)
    z_ref[...] = (jnp.dot(r, wp_ref[...], preferred_element_type=jnp.float32)
                  + bp_ref[...])


def _row_spec(width):
    return pl.BlockSpec((BM, width), lambda i: (i, 0))


def _pair_spec(width):
    return pl.BlockSpec((2, BM, width), lambda i: (0, i, 0))


def _full_spec(shape):
    return pl.BlockSpec(shape, lambda i: tuple(0 for _ in shape))


_GRID = (NPAD // BM,)
_CS = jax.ShapeDtypeStruct((NC, NPAD, HHALF), jnp.float32)

_tc1 = pl.pallas_call(
    _tc1_body,
    grid=_GRID,
    in_specs=[_row_spec(IN_DIM), _full_spec((IN_DIM, HID)), _pair_spec(16)],
    out_specs=_pair_spec(HHALF),
    out_shape=_CS,
)

_tc2 = pl.pallas_call(
    _tc2_body,
    grid=_GRID,
    in_specs=[_pair_spec(HHALF), _pair_spec(HHALF), _pair_spec(16),
              _full_spec((HID, HID)), _full_spec((1, HID))],
    out_specs=_pair_spec(HHALF),
    out_shape=_CS,
)

_tc3 = pl.pallas_call(
    _tc3_body,
    grid=_GRID,
    in_specs=[_pair_spec(HHALF), _pair_spec(HHALF), _pair_spec(16),
              _full_spec((HID, HID)), _full_spec((1, HID)),
              _full_spec((1, HID))],
    out_specs=_row_spec(HID),
    out_shape=jax.ShapeDtypeStruct((NPAD, HID), jnp.float32),
)


def kernel(x, edge_index, W1, b1, W2, b2, Wp, bp):
    row = edge_index[0].astype(jnp.int32)
    col = edge_index[1].astype(jnp.int32)

    ridx = jnp.concatenate(
        [row, jnp.full((EPAD - E,), N, jnp.int32)]).reshape(NS, MC, CHUNK)
    cidx = jnp.concatenate(
        [col, jnp.full((EPAD - E,), NPAD - 1, jnp.int32)]).reshape(NS, MC, CHUNK)
    dcol = jnp.concatenate(
        [col, jnp.arange(N, dtype=jnp.int32),
         jnp.full((EDPAD - ED,), NPAD - 1, jnp.int32)]).reshape(NW, DC, CHUNK)

    x_pad = jnp.pad(x, ((0, NPAD - N), (0, 0)))
    ones16 = jnp.ones((CHUNK, 16), jnp.float32)
    zeros16 = jnp.zeros((ROWS_PER_TILE, 16), jnp.float32)
    zeros32 = jnp.zeros((ROWS_PER_TILE, HHALF), jnp.float32)
    b1r = b1.reshape(1, HID)
    b2r = b2.reshape(1, HID)
    bpr = bp.reshape(1, HID)

    degp = _deg_kernel(dcol, ones16, zeros16)

    g1 = _tc1(x_pad, W1, degp)
    s1 = _edge_kernel(g1, ridx, cidx, zeros32)
    g2 = _tc2(s1, g1, degp, W2, b1r)
    s2 = _edge_kernel(g2, ridx, cidx, zeros32)
    z = _tc3(s2, g2, degp, Wp, b2r, bpr)
    return z[:N]


# speedup vs baseline: 1.1082x; 1.1082x over previous
import functools

import jax
import jax.numpy as jnp
from jax import lax
from jax.experimental import pallas as pl
from jax.experimental.pallas import tpu as pltpu
from jax.experimental.pallas import tpu_sc as plsc

N = 10000
NPAD = 10240
IN_DIM = 128
HID = 64
HHALF = HID // 2

NC = 2
NS = 16
NW = NC * NS
CHUNK = 128
ROWS_PER_TILE = NPAD // NS

E = 320000
EPT = E // NS
MC = 160
EFILL = MC * CHUNK - EPT
EPD = E // NW
DC = 79
DFILL = DC * CHUNK - EPD

_MESH = plsc.VectorSubcoreMesh(core_axis_name="c", subcore_axis_name="s")
_SC_PARAMS = pltpu.CompilerParams(use_tc_tiling_on_sc=False)

BM = 1024
BM3 = 1000


def _deg_body(col_hbm, ones_hbm, zeros_hbm, dfill_hbm, out_hbm, idx_v, src_v,
              ztile_v, acc_sh, sem):
    c = lax.axis_index("c")
    s = lax.axis_index("s")
    w = s * NC + c
    pltpu.sync_copy(ones_hbm, src_v)
    pltpu.sync_copy(zeros_hbm, ztile_v)
    pltpu.sync_copy(col_hbm.at[pl.ds(w * EPD, EPD)], idx_v.at[pl.ds(0, EPD)])
    pltpu.sync_copy(dfill_hbm, idx_v.at[pl.ds(EPD, DFILL)])
    pltpu.sync_copy(ztile_v, acc_sh.at[pl.ds(s * ROWS_PER_TILE, ROWS_PER_TILE)])
    plsc.subcore_barrier()

    def fire(j, carry):
        pltpu.async_copy(src_v, acc_sh.at[idx_v.at[pl.ds(j * CHUNK, CHUNK)]],
                         sem, add=True)
        return carry

    lax.fori_loop(0, DC, fire, 0)

    def drain(j, carry):
        pltpu.make_async_copy(src_v, acc_sh.at[idx_v.at[pl.ds(0, CHUNK)]],
                              sem).wait()
        return carry

    lax.fori_loop(0, DC, drain, 0)
    plsc.subcore_barrier()
    pltpu.sync_copy(acc_sh.at[pl.ds(s * ROWS_PER_TILE, ROWS_PER_TILE)],
                    out_hbm.at[c, pl.ds(s * ROWS_PER_TILE, ROWS_PER_TILE)])


_deg_kernel = functools.partial(
    pl.kernel,
    out_type=jax.ShapeDtypeStruct((NC, NPAD, 16), jnp.float32),
    mesh=_MESH,
    compiler_params=_SC_PARAMS,
    scratch_types=[
        pltpu.VMEM((DC * CHUNK,), jnp.int32),
        pltpu.VMEM((CHUNK, 16), jnp.float32),
        pltpu.VMEM((ROWS_PER_TILE, 16), jnp.float32),
        pltpu.VMEM_SHARED((NPAD, 16), jnp.float32),
        pltpu.SemaphoreType.DMA,
    ],
)(_deg_body)


def _edge_body(g_hbm, row_hbm, col_hbm, zeros_hbm, rfill_hbm, cfill_hbm,
               out_hbm, ridx_v, cidx_v, bufs, ztile_v, g_sh, acc_sh, sems):
    c = lax.axis_index("c")
    s = lax.axis_index("s")
    pltpu.sync_copy(row_hbm.at[pl.ds(s * EPT, EPT)], ridx_v.at[pl.ds(0, EPT)])
    pltpu.sync_copy(rfill_hbm, ridx_v.at[pl.ds(EPT, EFILL)])
    pltpu.sync_copy(col_hbm.at[pl.ds(s * EPT, EPT)], cidx_v.at[pl.ds(0, EPT)])
    pltpu.sync_copy(cfill_hbm, cidx_v.at[pl.ds(EPT, EFILL)])
    pltpu.sync_copy(zeros_hbm, ztile_v)
    pltpu.sync_copy(ztile_v, acc_sh.at[pl.ds(s * ROWS_PER_TILE, ROWS_PER_TILE)])
    pltpu.sync_copy(g_hbm.at[c, pl.ds(s * ROWS_PER_TILE, ROWS_PER_TILE)],
                    ztile_v)
    pltpu.sync_copy(ztile_v, g_sh.at[pl.ds(s * ROWS_PER_TILE, ROWS_PER_TILE)])
    plsc.subcore_barrier()

    npair = MC // 2
    buf0, buf1 = bufs
    sem0, sem1 = sems
    pltpu.async_copy(g_sh.at[ridx_v.at[pl.ds(0, CHUNK)]], buf0, sem0)

    def body(i, carry):
        j0 = 2 * i * CHUNK
        pltpu.async_copy(g_sh.at[ridx_v.at[pl.ds(j0 + CHUNK, CHUNK)]], buf1,
                         sem1)
        pltpu.make_async_copy(g_sh.at[ridx_v.at[pl.ds(j0, CHUNK)]], buf0,
                              sem0).wait()
        pltpu.sync_copy(buf0, acc_sh.at[cidx_v.at[pl.ds(j0, CHUNK)]], add=True)

        @pl.when(i < npair - 1)
        def _():
            pltpu.async_copy(g_sh.at[ridx_v.at[pl.ds(j0 + 2 * CHUNK, CHUNK)]],
                             buf0, sem0)

        pltpu.make_async_copy(g_sh.at[ridx_v.at[pl.ds(j0 + CHUNK, CHUNK)]],
                              buf1, sem1).wait()
        pltpu.sync_copy(buf1, acc_sh.at[cidx_v.at[pl.ds(j0 + CHUNK, CHUNK)]],
                        add=True)
        return carry

    lax.fori_loop(0, npair, body, 0)
    plsc.subcore_barrier()
    pltpu.sync_copy(acc_sh.at[pl.ds(s * ROWS_PER_TILE, ROWS_PER_TILE)],
                    out_hbm.at[c, pl.ds(s * ROWS_PER_TILE, ROWS_PER_TILE)])


_edge_kernel = functools.partial(
    pl.kernel,
    out_type=jax.ShapeDtypeStruct((NC, NPAD, HHALF), jnp.float32),
    mesh=_MESH,
    compiler_params=_SC_PARAMS,
    scratch_types=[
        pltpu.VMEM((MC * CHUNK,), jnp.int32),
        pltpu.VMEM((MC * CHUNK,), jnp.int32),
        [pltpu.VMEM((CHUNK, HHALF), jnp.float32) for _ in range(2)],
        pltpu.VMEM((ROWS_PER_TILE, HHALF), jnp.float32),
        pltpu.VMEM_SHARED((NPAD, HHALF), jnp.float32),
        pltpu.VMEM_SHARED((NPAD, HHALF), jnp.float32),
        [pltpu.SemaphoreType.DMA for _ in range(2)],
    ],
)(_edge_body)


def _tc1_body(x_ref, w1_ref, degp_ref, g_ref, dinvb_ref):
    i = pl.program_id(0)
    rowid = lax.broadcasted_iota(jnp.int32, (BM, 1), 0) + i * BM
    deg = (degp_ref[0, :, 0:1] + degp_ref[1, :, 0:1]
           + jnp.where(rowid < N, 1.0, 0.0))
    dinv = jnp.where(deg > 0.0, lax.rsqrt(deg), 0.0)
    h = jnp.dot(x_ref[...], w1_ref[...], preferred_element_type=jnp.float32)
    g = h * dinv
    g_ref[0] = g[:, :HHALF]
    g_ref[1] = g[:, HHALF:]
    dinvb_ref[...] = jnp.broadcast_to(dinv, (BM, HID))


def _join_cols(ref):
    return jnp.concatenate([ref[0], ref[1]], axis=1)


def _tc2_body(s_ref, g1_ref, dinvb_ref, w2_ref, b1_ref, g2_ref):
    dinv = dinvb_ref[...]
    a1 = dinv * (_join_cols(s_ref) + _join_cols(g1_ref)) + b1_ref[...]
    r = jnp.maximum(a1, 0.0)
    h2 = jnp.dot(r, w2_ref[...], preferred_element_type=jnp.float32)
    g2 = h2 * dinv
    g2_ref[0] = g2[:, :HHALF]
    g2_ref[1] = g2[:, HHALF:]


def _tc3_body(s_ref, g2_ref, dinvb_ref, wp_ref, b2_ref, bp_ref, z_ref):
    dinv = dinvb_ref[...]
    a2 = dinv * (_join_cols(s_ref) + _join_cols(g2_ref)) + b2_ref[...]
    r = jnp.maximum(a2, 0.0)
    z_ref[...] = (jnp.dot(r, wp_ref[...], preferred_element_type=jnp.float32)
                  + bp_ref[...])


def _row_spec(width, bm=BM):
    return pl.BlockSpec((bm, width), lambda i: (i, 0))


def _pair_spec(width, bm=BM):
    return pl.BlockSpec((2, bm, width), lambda i: (0, i, 0))


def _full_spec(shape):
    return pl.BlockSpec(shape, lambda i: tuple(0 for _ in shape))


_CS = jax.ShapeDtypeStruct((NC, NPAD, HHALF), jnp.float32)

_tc1 = pl.pallas_call(
    _tc1_body,
    grid=(NPAD // BM,),
    in_specs=[_row_spec(IN_DIM), _full_spec((IN_DIM, HID)), _pair_spec(16)],
    out_specs=(_pair_spec(HHALF), _row_spec(HID)),
    out_shape=(_CS, jax.ShapeDtypeStruct((NPAD, HID), jnp.float32)),
)

_tc2 = pl.pallas_call(
    _tc2_body,
    grid=(NPAD // BM,),
    in_specs=[_pair_spec(HHALF), _pair_spec(HHALF), _row_spec(HID),
              _full_spec((HID, HID)), _full_spec((1, HID))],
    out_specs=_pair_spec(HHALF),
    out_shape=_CS,
)

_tc3 = pl.pallas_call(
    _tc3_body,
    grid=(N // BM3,),
    in_specs=[_pair_spec(HHALF, BM3), _pair_spec(HHALF, BM3),
              _row_spec(HID, BM3), _full_spec((HID, HID)),
              _full_spec((1, HID)), _full_spec((1, HID))],
    out_specs=_row_spec(HID, BM3),
    out_shape=jax.ShapeDtypeStruct((N, HID), jnp.float32),
)


def kernel(x, edge_index, W1, b1, W2, b2, Wp, bp):
    row = edge_index[0].astype(jnp.int32)
    col = edge_index[1].astype(jnp.int32)

    x_pad = jnp.pad(x, ((0, NPAD - N), (0, 0)))
    ones16 = jnp.ones((CHUNK, 16), jnp.float32)
    zeros16 = jnp.zeros((ROWS_PER_TILE, 16), jnp.float32)
    zeros32 = jnp.zeros((ROWS_PER_TILE, HHALF), jnp.float32)
    rfill = jnp.full((EFILL,), N, jnp.int32)
    cfill = jnp.full((EFILL,), NPAD - 1, jnp.int32)
    dfill = jnp.full((DFILL,), NPAD - 1, jnp.int32)
    b1r = b1.reshape(1, HID)
    b2r = b2.reshape(1, HID)
    bpr = bp.reshape(1, HID)

    degp = _deg_kernel(col, ones16, zeros16, dfill)

    g1, dinvb = _tc1(x_pad, W1, degp)
    s1 = _edge_kernel(g1, row, col, zeros32, rfill, cfill)
    g2 = _tc2(s1, g1, dinvb, W2, b1r)
    s2 = _edge_kernel(g2, row, col, zeros32, rfill, cfill)
    z = _tc3(s2, g2, dinvb, Wp, b2r, bpr)
    return z


# speedup vs baseline: 1.1515x; 1.0391x over previous
import functools

import jax
import jax.numpy as jnp
from jax import lax
from jax.experimental import pallas as pl
from jax.experimental.pallas import tpu as pltpu
from jax.experimental.pallas import tpu_sc as plsc

N = 10000
NPAD = 10240
IN_DIM = 128
HID = 64
HHALF = HID // 2

NC = 2
NS = 16
NW = NC * NS
CHUNK = 128
ROWS_PER_TILE = NPAD // NS

E = 320000
EPT = E // NS
MC = 160
EFILL = MC * CHUNK - EPT
EPD = E // NW
DC = 79
DFILL = DC * CHUNK - EPD

_MESH = plsc.VectorSubcoreMesh(core_axis_name="c", subcore_axis_name="s")
_SC_PARAMS = pltpu.CompilerParams(use_tc_tiling_on_sc=False)

BM = 1024
BM3 = 1000


def _deg_body(ei_hbm, ones_hbm, zeros_hbm, dfill_hbm, out_hbm, idx_v, src_v,
              ztile_v, acc_sh, sem):
    c = lax.axis_index("c")
    s = lax.axis_index("s")
    w = s * NC + c
    pltpu.sync_copy(ones_hbm, src_v)
    pltpu.sync_copy(zeros_hbm, ztile_v)
    pltpu.sync_copy(ei_hbm.at[1, pl.ds(w * EPD, EPD)], idx_v.at[pl.ds(0, EPD)])
    pltpu.sync_copy(dfill_hbm, idx_v.at[pl.ds(EPD, DFILL)])
    pltpu.sync_copy(ztile_v, acc_sh.at[pl.ds(s * ROWS_PER_TILE, ROWS_PER_TILE)])
    plsc.subcore_barrier()

    def fire(j, carry):
        pltpu.async_copy(src_v, acc_sh.at[idx_v.at[pl.ds(j * CHUNK, CHUNK)]],
                         sem, add=True)
        return carry

    lax.fori_loop(0, DC, fire, 0)

    def drain(j, carry):
        pltpu.make_async_copy(src_v, acc_sh.at[idx_v.at[pl.ds(0, CHUNK)]],
                              sem).wait()
        return carry

    lax.fori_loop(0, DC, drain, 0)
    plsc.subcore_barrier()
    pltpu.sync_copy(acc_sh.at[pl.ds(s * ROWS_PER_TILE, ROWS_PER_TILE)],
                    out_hbm.at[c, pl.ds(s * ROWS_PER_TILE, ROWS_PER_TILE)])


_deg_kernel = functools.partial(
    pl.kernel,
    out_type=jax.ShapeDtypeStruct((NC, NPAD, 16), jnp.float32),
    mesh=_MESH,
    compiler_params=_SC_PARAMS,
    scratch_types=[
        pltpu.VMEM((DC * CHUNK,), jnp.int32),
        pltpu.VMEM((CHUNK, 16), jnp.float32),
        pltpu.VMEM((ROWS_PER_TILE, 16), jnp.float32),
        pltpu.VMEM_SHARED((NPAD, 16), jnp.float32),
        pltpu.SemaphoreType.DMA,
    ],
)(_deg_body)


def _edge_body(g_hbm, ei_hbm, zeros_hbm, rfill_hbm, cfill_hbm,
               out_hbm, ridx_v, cidx_v, bufs, ztile_v, g_sh, acc_sh, sems):
    c = lax.axis_index("c")
    s = lax.axis_index("s")
    pltpu.sync_copy(ei_hbm.at[0, pl.ds(s * EPT, EPT)], ridx_v.at[pl.ds(0, EPT)])
    pltpu.sync_copy(rfill_hbm, ridx_v.at[pl.ds(EPT, EFILL)])
    pltpu.sync_copy(ei_hbm.at[1, pl.ds(s * EPT, EPT)], cidx_v.at[pl.ds(0, EPT)])
    pltpu.sync_copy(cfill_hbm, cidx_v.at[pl.ds(EPT, EFILL)])
    pltpu.sync_copy(zeros_hbm, ztile_v)
    pltpu.sync_copy(ztile_v, acc_sh.at[pl.ds(s * ROWS_PER_TILE, ROWS_PER_TILE)])
    pltpu.sync_copy(g_hbm.at[c, pl.ds(s * ROWS_PER_TILE, ROWS_PER_TILE)],
                    ztile_v)
    pltpu.sync_copy(ztile_v, g_sh.at[pl.ds(s * ROWS_PER_TILE, ROWS_PER_TILE)])
    plsc.subcore_barrier()

    npair = MC // 2
    buf0, buf1 = bufs
    sem0, sem1 = sems
    pltpu.async_copy(g_sh.at[ridx_v.at[pl.ds(0, CHUNK)]], buf0, sem0)

    def body(i, carry):
        j0 = 2 * i * CHUNK
        pltpu.async_copy(g_sh.at[ridx_v.at[pl.ds(j0 + CHUNK, CHUNK)]], buf1,
                         sem1)
        pltpu.make_async_copy(g_sh.at[ridx_v.at[pl.ds(j0, CHUNK)]], buf0,
                              sem0).wait()
        pltpu.sync_copy(buf0, acc_sh.at[cidx_v.at[pl.ds(j0, CHUNK)]], add=True)

        @pl.when(i < npair - 1)
        def _():
            pltpu.async_copy(g_sh.at[ridx_v.at[pl.ds(j0 + 2 * CHUNK, CHUNK)]],
                             buf0, sem0)

        pltpu.make_async_copy(g_sh.at[ridx_v.at[pl.ds(j0 + CHUNK, CHUNK)]],
                              buf1, sem1).wait()
        pltpu.sync_copy(buf1, acc_sh.at[cidx_v.at[pl.ds(j0 + CHUNK, CHUNK)]],
                        add=True)
        return carry

    lax.fori_loop(0, npair, body, 0)
    plsc.subcore_barrier()
    pltpu.sync_copy(acc_sh.at[pl.ds(s * ROWS_PER_TILE, ROWS_PER_TILE)],
                    out_hbm.at[c, pl.ds(s * ROWS_PER_TILE, ROWS_PER_TILE)])


_edge_kernel = functools.partial(
    pl.kernel,
    out_type=jax.ShapeDtypeStruct((NC, NPAD, HHALF), jnp.float32),
    mesh=_MESH,
    compiler_params=_SC_PARAMS,
    scratch_types=[
        pltpu.VMEM((MC * CHUNK,), jnp.int32),
        pltpu.VMEM((MC * CHUNK,), jnp.int32),
        [pltpu.VMEM((CHUNK, HHALF), jnp.float32) for _ in range(2)],
        pltpu.VMEM((ROWS_PER_TILE, HHALF), jnp.float32),
        pltpu.VMEM_SHARED((NPAD, HHALF), jnp.float32),
        pltpu.VMEM_SHARED((NPAD, HHALF), jnp.float32),
        [pltpu.SemaphoreType.DMA for _ in range(2)],
    ],
)(_edge_body)


def _tc1_body(x_ref, w1_ref, degp_ref, g_ref, dinvb_ref):
    i = pl.program_id(0)
    rowid = lax.broadcasted_iota(jnp.int32, (BM, 1), 0) + i * BM
    deg = (degp_ref[0, :, 0:1] + degp_ref[1, :, 0:1]
           + jnp.where(rowid < N, 1.0, 0.0))
    dinv = jnp.where(deg > 0.0, lax.rsqrt(deg), 0.0)
    h = jnp.dot(x_ref[...], w1_ref[...], preferred_element_type=jnp.float32)
    g = h * dinv
    g_ref[0] = g[:, :HHALF]
    g_ref[1] = g[:, HHALF:]
    dinvb_ref[...] = jnp.broadcast_to(dinv, (BM, HID))


def _join_cols(ref):
    return jnp.concatenate([ref[0], ref[1]], axis=1)


def _tc2_body(s_ref, g1_ref, dinvb_ref, w2_ref, b1_ref, g2_ref):
    dinv = dinvb_ref[...]
    a1 = dinv * (_join_cols(s_ref) + _join_cols(g1_ref)) + b1_ref[...]
    r = jnp.maximum(a1, 0.0)
    h2 = jnp.dot(r, w2_ref[...], preferred_element_type=jnp.float32)
    g2 = h2 * dinv
    g2_ref[0] = g2[:, :HHALF]
    g2_ref[1] = g2[:, HHALF:]


def _tc3_body(s_ref, g2_ref, dinvb_ref, wp_ref, b2_ref, bp_ref, z_ref):
    dinv = dinvb_ref[...]
    a2 = dinv * (_join_cols(s_ref) + _join_cols(g2_ref)) + b2_ref[...]
    r = jnp.maximum(a2, 0.0)
    z_ref[...] = (jnp.dot(r, wp_ref[...], preferred_element_type=jnp.float32)
                  + bp_ref[...])


def _row_spec(width, bm=BM):
    return pl.BlockSpec((bm, width), lambda i: (i, 0))


def _pair_spec(width, bm=BM):
    return pl.BlockSpec((2, bm, width), lambda i: (0, i, 0))


def _full_spec(shape):
    return pl.BlockSpec(shape, lambda i: tuple(0 for _ in shape))


_CS = jax.ShapeDtypeStruct((NC, NPAD, HHALF), jnp.float32)

_tc1 = pl.pallas_call(
    _tc1_body,
    grid=(NPAD // BM,),
    in_specs=[_row_spec(IN_DIM), _full_spec((IN_DIM, HID)), _pair_spec(16)],
    out_specs=(_pair_spec(HHALF), _row_spec(HID)),
    out_shape=(_CS, jax.ShapeDtypeStruct((NPAD, HID), jnp.float32)),
)

_tc2 = pl.pallas_call(
    _tc2_body,
    grid=(NPAD // BM,),
    in_specs=[_pair_spec(HHALF), _pair_spec(HHALF), _row_spec(HID),
              _full_spec((HID, HID)), _full_spec((1, HID))],
    out_specs=_pair_spec(HHALF),
    out_shape=_CS,
)

_tc3 = pl.pallas_call(
    _tc3_body,
    grid=(N // BM3,),
    in_specs=[_pair_spec(HHALF, BM3), _pair_spec(HHALF, BM3),
              _row_spec(HID, BM3), _full_spec((HID, HID)),
              _full_spec((1, HID)), _full_spec((1, HID))],
    out_specs=_row_spec(HID, BM3),
    out_shape=jax.ShapeDtypeStruct((N, HID), jnp.float32),
)


def kernel(x, edge_index, W1, b1, W2, b2, Wp, bp):
    ei = edge_index.astype(jnp.int32)

    x_pad = jnp.pad(x, ((0, NPAD - N), (0, 0)))
    ones16 = jnp.ones((CHUNK, 16), jnp.float32)
    zeros16 = jnp.zeros((ROWS_PER_TILE, 16), jnp.float32)
    zeros32 = jnp.zeros((ROWS_PER_TILE, HHALF), jnp.float32)
    rfill = jnp.full((EFILL,), N, jnp.int32)
    cfill = jnp.full((EFILL,), NPAD - 1, jnp.int32)
    dfill = jnp.full((DFILL,), NPAD - 1, jnp.int32)
    b1r = b1.reshape(1, HID)
    b2r = b2.reshape(1, HID)
    bpr = bp.reshape(1, HID)

    degp = _deg_kernel(ei, ones16, zeros16, dfill)

    g1, dinvb = _tc1(x_pad, W1, degp)
    s1 = _edge_kernel(g1, ei, zeros32, rfill, cfill)
    g2 = _tc2(s1, g1, dinvb, W2, b1r)
    s2 = _edge_kernel(g2, ei, zeros32, rfill, cfill)
    z = _tc3(s2, g2, dinvb, Wp, b2r, bpr)
    return z


# speedup vs baseline: 1.2808x; 1.1123x over previous
import functools

import jax
import jax.numpy as jnp
from jax import lax
from jax.experimental import pallas as pl
from jax.experimental.pallas import tpu as pltpu
from jax.experimental.pallas import tpu_sc as plsc

N = 10000
NPAD = 10240
IN_DIM = 128
HID = 64
HHALF = HID // 2

NC = 2
NS = 16
NW = NC * NS
CHUNK = 128
ROWS_PER_TILE = NPAD // NS

E = 320000
EPT = E // NS
MC = 160
EFILL = MC * CHUNK - EPT
EPD = E // NW
DC = 79
DFILL = DC * CHUNK - EPD

_MESH = plsc.VectorSubcoreMesh(core_axis_name="c", subcore_axis_name="s")
_SC_PARAMS = pltpu.CompilerParams(use_tc_tiling_on_sc=False)

BM = 1024
BM3 = 1000


def _deg_body(ei_hbm, ones_hbm, zeros_hbm, dfill_hbm, out_hbm, idx_v, src_v,
              ztile_v, acc_sh, sem):
    c = lax.axis_index("c")
    s = lax.axis_index("s")
    w = s * NC + c
    pltpu.sync_copy(ones_hbm, src_v)
    pltpu.sync_copy(zeros_hbm, ztile_v)
    pltpu.sync_copy(ei_hbm.at[1, pl.ds(w * EPD, EPD)], idx_v.at[pl.ds(0, EPD)])
    pltpu.sync_copy(dfill_hbm, idx_v.at[pl.ds(EPD, DFILL)])
    pltpu.sync_copy(ztile_v, acc_sh.at[pl.ds(s * ROWS_PER_TILE, ROWS_PER_TILE)])
    plsc.subcore_barrier()

    def fire(j, carry):
        pltpu.async_copy(src_v, acc_sh.at[idx_v.at[pl.ds(j * CHUNK, CHUNK)]],
                         sem, add=True)
        return carry

    lax.fori_loop(0, DC, fire, 0)

    def drain(j, carry):
        pltpu.make_async_copy(src_v, acc_sh.at[idx_v.at[pl.ds(0, CHUNK)]],
                              sem).wait()
        return carry

    lax.fori_loop(0, DC, drain, 0)
    plsc.subcore_barrier()
    pltpu.sync_copy(acc_sh.at[pl.ds(s * ROWS_PER_TILE, ROWS_PER_TILE)],
                    out_hbm.at[c, pl.ds(s * ROWS_PER_TILE, ROWS_PER_TILE)])


_deg_kernel = functools.partial(
    pl.kernel,
    out_type=jax.ShapeDtypeStruct((NC, NPAD, 16), jnp.float32),
    mesh=_MESH,
    compiler_params=_SC_PARAMS,
    scratch_types=[
        pltpu.VMEM((DC * CHUNK,), jnp.int32),
        pltpu.VMEM((CHUNK, 16), jnp.float32),
        pltpu.VMEM((ROWS_PER_TILE, 16), jnp.float32),
        pltpu.VMEM_SHARED((NPAD, 16), jnp.float32),
        pltpu.SemaphoreType.DMA,
    ],
)(_deg_body)


def _edge_body(q_hbm, ei_hbm, zeros_hbm, rfill_hbm, cfill_hbm,
               out_hbm, ridx_v, cidx_v, bufs, ztile_v, g_sh, acc_sh, sems):
    c = lax.axis_index("c")
    s = lax.axis_index("s")
    pltpu.sync_copy(ei_hbm.at[0, pl.ds(s * EPT, EPT)], ridx_v.at[pl.ds(0, EPT)])
    pltpu.sync_copy(rfill_hbm, ridx_v.at[pl.ds(EPT, EFILL)])
    pltpu.sync_copy(ei_hbm.at[1, pl.ds(s * EPT, EPT)], cidx_v.at[pl.ds(0, EPT)])
    pltpu.sync_copy(cfill_hbm, cidx_v.at[pl.ds(EPT, EFILL)])
    pltpu.sync_copy(zeros_hbm, ztile_v)
    pltpu.sync_copy(ztile_v, acc_sh.at[pl.ds(s * ROWS_PER_TILE, ROWS_PER_TILE)])
    pltpu.sync_copy(q_hbm.at[pl.ds(s * ROWS_PER_TILE, ROWS_PER_TILE),
                             pl.ds(HHALF * c, HHALF)],
                    ztile_v)
    pltpu.sync_copy(ztile_v, g_sh.at[pl.ds(s * ROWS_PER_TILE, ROWS_PER_TILE)])
    plsc.subcore_barrier()

    npair = MC // 2
    buf0, buf1 = bufs
    sem0, sem1 = sems
    pltpu.async_copy(g_sh.at[ridx_v.at[pl.ds(0, CHUNK)]], buf0, sem0)

    def body(i, carry):
        j0 = 2 * i * CHUNK
        pltpu.async_copy(g_sh.at[ridx_v.at[pl.ds(j0 + CHUNK, CHUNK)]], buf1,
                         sem1)
        pltpu.make_async_copy(g_sh.at[ridx_v.at[pl.ds(j0, CHUNK)]], buf0,
                              sem0).wait()
        pltpu.sync_copy(buf0, acc_sh.at[cidx_v.at[pl.ds(j0, CHUNK)]], add=True)

        @pl.when(i < npair - 1)
        def _():
            pltpu.async_copy(g_sh.at[ridx_v.at[pl.ds(j0 + 2 * CHUNK, CHUNK)]],
                             buf0, sem0)

        pltpu.make_async_copy(g_sh.at[ridx_v.at[pl.ds(j0 + CHUNK, CHUNK)]],
                              buf1, sem1).wait()
        pltpu.sync_copy(buf1, acc_sh.at[cidx_v.at[pl.ds(j0 + CHUNK, CHUNK)]],
                        add=True)
        return carry

    lax.fori_loop(0, npair, body, 0)
    plsc.subcore_barrier()
    pltpu.sync_copy(acc_sh.at[pl.ds(s * ROWS_PER_TILE, ROWS_PER_TILE)],
                    out_hbm.at[pl.ds(s * ROWS_PER_TILE, ROWS_PER_TILE),
                               pl.ds(HHALF * c, HHALF)])


_edge_kernel = functools.partial(
    pl.kernel,
    out_type=jax.ShapeDtypeStruct((NPAD, HID), jnp.float32),
    mesh=_MESH,
    compiler_params=_SC_PARAMS,
    scratch_types=[
        pltpu.VMEM((MC * CHUNK,), jnp.int32),
        pltpu.VMEM((MC * CHUNK,), jnp.int32),
        [pltpu.VMEM((CHUNK, HHALF), jnp.float32) for _ in range(2)],
        pltpu.VMEM((ROWS_PER_TILE, HHALF), jnp.float32),
        pltpu.VMEM_SHARED((NPAD, HHALF), jnp.float32),
        pltpu.VMEM_SHARED((NPAD, HHALF), jnp.float32),
        [pltpu.SemaphoreType.DMA for _ in range(2)],
    ],
)(_edge_body)


def _tc1_body(x_ref, w1_ref, degp_ref, q_ref):
    i = pl.program_id(0)
    rowid = lax.broadcasted_iota(jnp.int32, (BM, 1), 0) + i * BM
    deg = (degp_ref[0, :, 0:1] + degp_ref[1, :, 0:1]
           + jnp.where(rowid < N, 1.0, 0.0))
    dinv = jnp.where(deg > 0.0, lax.rsqrt(deg), 0.0)
    h = jnp.dot(x_ref[...], w1_ref[...], preferred_element_type=jnp.float32)
    q_ref[...] = jnp.concatenate(
        [h * dinv, jnp.broadcast_to(dinv, (BM, HID))], axis=1)


def _tc2_body(s_ref, q1_ref, w2_ref, b1_ref, q2_ref):
    g1 = q1_ref[:, :HID]
    dinv = q1_ref[:, HID:]
    a1 = dinv * (s_ref[...] + g1) + b1_ref[...]
    r = jnp.maximum(a1, 0.0)
    h2 = jnp.dot(r, w2_ref[...], preferred_element_type=jnp.float32)
    q2_ref[...] = jnp.concatenate([h2 * dinv, dinv], axis=1)


def _tc3_body(s_ref, q2_ref, wp_ref, b2_ref, bp_ref, z_ref):
    g2 = q2_ref[:, :HID]
    dinv = q2_ref[:, HID:]
    a2 = dinv * (s_ref[...] + g2) + b2_ref[...]
    r = jnp.maximum(a2, 0.0)
    z_ref[...] = (jnp.dot(r, wp_ref[...], preferred_element_type=jnp.float32)
                  + bp_ref[...])


def _row_spec(width, bm=BM):
    return pl.BlockSpec((bm, width), lambda i: (i, 0))


def _pair_spec(width, bm=BM):
    return pl.BlockSpec((2, bm, width), lambda i: (0, i, 0))


def _full_spec(shape):
    return pl.BlockSpec(shape, lambda i: tuple(0 for _ in shape))


_QS = jax.ShapeDtypeStruct((NPAD, 2 * HID), jnp.float32)

_tc1 = pl.pallas_call(
    _tc1_body,
    grid=(NPAD // BM,),
    in_specs=[_row_spec(IN_DIM), _full_spec((IN_DIM, HID)), _pair_spec(16)],
    out_specs=_row_spec(2 * HID),
    out_shape=_QS,
)

_tc2 = pl.pallas_call(
    _tc2_body,
    grid=(NPAD // BM,),
    in_specs=[_row_spec(HID), _row_spec(2 * HID),
              _full_spec((HID, HID)), _full_spec((1, HID))],
    out_specs=_row_spec(2 * HID),
    out_shape=_QS,
)

_tc3 = pl.pallas_call(
    _tc3_body,
    grid=(N // BM3,),
    in_specs=[_row_spec(HID, BM3), _row_spec(2 * HID, BM3),
              _full_spec((HID, HID)), _full_spec((1, HID)),
              _full_spec((1, HID))],
    out_specs=_row_spec(HID, BM3),
    out_shape=jax.ShapeDtypeStruct((N, HID), jnp.float32),
)


def kernel(x, edge_index, W1, b1, W2, b2, Wp, bp):
    ei = edge_index.astype(jnp.int32)

    x_pad = jnp.pad(x, ((0, NPAD - N), (0, 0)))
    ones16 = jnp.ones((CHUNK, 16), jnp.float32)
    zeros16 = jnp.zeros((ROWS_PER_TILE, 16), jnp.float32)
    zeros32 = jnp.zeros((ROWS_PER_TILE, HHALF), jnp.float32)
    rfill = jnp.full((EFILL,), N, jnp.int32)
    cfill = jnp.full((EFILL,), NPAD - 1, jnp.int32)
    dfill = jnp.full((DFILL,), NPAD - 1, jnp.int32)
    b1r = b1.reshape(1, HID)
    b2r = b2.reshape(1, HID)
    bpr = bp.reshape(1, HID)

    degp = _deg_kernel(ei, ones16, zeros16, dfill)

    q1 = _tc1(x_pad, W1, degp)
    s1 = _edge_kernel(q1, ei, zeros32, rfill, cfill)
    q2 = _tc2(s1, q1, W2, b1r)
    s2 = _edge_kernel(q2, ei, zeros32, rfill, cfill)
    z = _tc3(s2, q2, Wp, b2r, bpr)
    return z
